# baseline (speedup 1.0000x reference)
import functools

import jax
import jax.numpy as jnp
from jax import lax
from jax.experimental import pallas as pl
from jax.experimental.pallas import tpu as pltpu
from jax.experimental.pallas import tpu_sc as plsc

N = 50000
NP = 50176
E = 800000
D = 64
H = 128
W = 32

NC = 2
NS = 16
C = 80
EPT = E // NS
NITER = EPT // C
RPT = NP // NS
DR = 112
NDR = RPT // DR

CW = 8
CC_ = 40
CK = 5
CEPW = E // (NC * NS)
CNG = CEPW // (CC_ * CK)

K = 8
NG = (NITER - 1) // K

BN = 784
GRID = NP // BN

_mesh = lambda: plsc.VectorSubcoreMesh(core_axis_name="c", subcore_axis_name="s")


def _counts(dst, z8, o8):
    @functools.partial(
        pl.kernel,
        mesh=_mesh(),
        compiler_params=pltpu.CompilerParams(use_tc_tiling_on_sc=False),
        out_type=jax.ShapeDtypeStruct((NC, NP, CW), jnp.float32),
        scratch_types=[
            pltpu.VMEM((CK, CC_), jnp.int32),
            pltpu.VMEM((CC_, CW), jnp.float32),
            pltpu.VMEM((DR, CW), jnp.float32),
            pltpu.VMEM_SHARED((NP, CW), jnp.float32),
            pltpu.SemaphoreType.DMA,
        ],
    )
    def k(dstc_h, z8_h, o8_h, cnt_h, cidx, ones, cbuf, cacc, ssem):
        c = lax.axis_index("c")
        s = lax.axis_index("s")
        pltpu.sync_copy(o8_h, ones)
        pltpu.sync_copy(z8_h, cbuf)
        for j in range(NDR):
            pltpu.sync_copy(cbuf, cacc.at[pl.ds(s * RPT + j * DR, DR)])
        plsc.subcore_barrier()
        for cc in range(NC):
            @pl.when(c == cc)
            def _():
                def body(g, carry):
                    pltpu.sync_copy(dstc_h.at[s, cc, g], cidx)
                    sds = [pltpu.async_copy(ones, cacc.at[cidx.at[j]],
                                            ssem, add=True)
                           for j in range(CK)]
                    for d in sds:
                        d.wait()
                    return carry

                lax.fori_loop(0, CNG, body, 0)
                plsc.subcore_barrier()
                for j in range(NDR):
                    r0 = s * RPT + j * DR
                    pltpu.sync_copy(cacc.at[pl.ds(r0, DR)], cbuf)
                    pltpu.sync_copy(cbuf, cnt_h.at[cc, pl.ds(r0, DR)])

    return k(dst.reshape(NS, NC, CNG, CK, CC_), z8, o8)


def _agg(table, F, nchunks, src, dst, z32):
    rounds = nchunks // NC

    @functools.partial(
        pl.kernel,
        mesh=_mesh(),
        compiler_params=pltpu.CompilerParams(use_tc_tiling_on_sc=False),
        out_type=jax.ShapeDtypeStruct((NP, H), jnp.bfloat16),
        scratch_types=[
            pltpu.VMEM((2, K, C), jnp.int32),
            pltpu.VMEM((2, K, C), jnp.int32),
            pltpu.VMEM((K, C), jnp.int32),
            pltpu.VMEM((K, C, W), jnp.bfloat16),
            pltpu.VMEM((DR, W), jnp.bfloat16),
            pltpu.VMEM_SHARED((NP, W), jnp.bfloat16),
            pltpu.SemaphoreType.DMA((K,)),
            pltpu.SemaphoreType.DMA((K,)),
        ],
    )
    def k(tbl_h, src_h, dst_h, z32_h, out_h,
          sidx2, didx2, sadj, rows2, dbuf, acc, gsem, ssem):
        c = lax.axis_index("c")
        s = lax.axis_index("s")

        def adjust(p, ci):
            for j in range(K):
                for u in range(C // 16):
                    v = sidx2[p, j, pl.ds(u * 16, 16)]
                    sadj[j, pl.ds(u * 16, 16)] = v * F + ci

        def wait_slot(j, prev_p):
            pltpu.make_async_copy(rows2.at[j], acc.at[didx2.at[prev_p, j]],
                                  ssem.at[j]).wait()

        for cc in range(NC):
            @pl.when(c == cc)
            def _():
                for rnd in range(rounds):
                    ci = cc + NC * rnd
                    pltpu.sync_copy(z32_h, dbuf)
                    for j in range(NDR):
                        pltpu.sync_copy(dbuf, acc.at[pl.ds(s * RPT + j * DR, DR)])
                    plsc.subcore_barrier()

                    def gbody(g, p, first):
                        pltpu.sync_copy(src_h.at[s, pl.ds(K * g, K)],
                                        sidx2.at[p])
                        pltpu.sync_copy(dst_h.at[s, pl.ds(K * g, K)],
                                        didx2.at[p])
                        adjust(p, ci)
                        gds = []
                        for j in range(K):
                            if not first:
                                wait_slot(j, 1 - p)
                            gds.append(pltpu.async_copy(
                                tbl_h.at[sadj.at[j]], rows2.at[j], gsem.at[j]))
                        for j in range(K):
                            gds[j].wait()
                            pltpu.async_copy(rows2.at[j],
                                             acc.at[didx2.at[p, j]],
                                             ssem.at[j], add=True)

                    gbody(0, 0, True)

                    def pair(t, carry):
                        gbody(2 * t + 1, 1, False)
                        gbody(2 * t + 2, 0, False)
                        return carry

                    lax.fori_loop(0, (NG - 2) // 2, pair, 0)
                    gbody(NG - 1, 1, False)
                    for j in range(K):
                        wait_slot(j, 1)
                    pltpu.sync_copy(src_h.at[s, pl.ds(K * NG, 1)],
                                    sidx2.at[0, pl.ds(0, 1)])
                    pltpu.sync_copy(dst_h.at[s, pl.ds(K * NG, 1)],
                                    didx2.at[0, pl.ds(0, 1)])
                    for u in range(C // 16):
                        v = sidx2[0, 0, pl.ds(u * 16, 16)]
                        sadj[0, pl.ds(u * 16, 16)] = v * F + ci
                    pltpu.async_copy(tbl_h.at[sadj.at[0]], rows2.at[0],
                                     gsem.at[0]).wait()
                    pltpu.sync_copy(rows2.at[0], acc.at[didx2.at[0, 0]],
                                    add=True)
                    plsc.subcore_barrier()
                    for j in range(NDR):
                        r0 = s * RPT + j * DR
                        pltpu.sync_copy(acc.at[pl.ds(r0, DR)], dbuf)
                        pltpu.sync_copy(
                            dbuf, out_h.at[pl.ds(r0, DR), pl.ds(ci * W, W)])

    return k(table, src.reshape(NS, NITER, C), dst.reshape(NS, NITER, C), z32)


def _dense1(af, c0, c1, x, W1l, b1l, W1r):
    def body(af_r, c0_r, c1_r, x_r, wl_r, bl_r, wr_r, o):
        cnt = c0_r[:, 0:1] + c1_r[:, 0:1]
        recip = 1.0 / jnp.maximum(cnt, 1.0)
        m = lax.dot_general(af_r[:, :D].astype(jnp.float32), wl_r[...],
                            (((1,), (1,)), ((), ())),
                            preferred_element_type=jnp.float32)
        sf = lax.dot_general(x_r[...], wr_r[...],
                             (((1,), (1,)), ((), ())),
                             preferred_element_type=jnp.float32)
        o[...] = jnp.maximum(m * recip + bl_r[...] + sf, 0.0).astype(jnp.bfloat16)

    node = lambda w: pl.BlockSpec((BN, w), lambda i: (i, 0))
    full = lambda a, b: pl.BlockSpec((a, b), lambda i: (0, 0))
    return pl.pallas_call(
        body,
        grid=(GRID,),
        in_specs=[node(H), node(CW), node(CW), node(D),
                  full(H, D), full(1, H), full(H, D)],
        out_specs=node(H),
        out_shape=jax.ShapeDtypeStruct((NP, H), jnp.bfloat16),
    )(af, c0, c1, x, W1l, b1l, W1r)


def _dense2(gf, hf, c0, c1, W2l, b2l, W2r, Wc, bc):
    def body(gf_r, hf_r, c0_r, c1_r, wl_r, bl_r, wr_r, wc_r, bc_r, o):
        cnt = c0_r[:, 0:1] + c1_r[:, 0:1]
        recip = 1.0 / jnp.maximum(cnt, 1.0)
        m = lax.dot_general(gf_r[...].astype(jnp.float32), wl_r[...],
                            (((1,), (1,)), ((), ())),
                            preferred_element_type=jnp.float32)
        sf = lax.dot_general(hf_r[...].astype(jnp.float32), wr_r[...],
                             (((1,), (1,)), ((), ())),
                             preferred_element_type=jnp.float32)
        h = jnp.maximum(m * recip + bl_r[...] + sf, 0.0)
        logit = jnp.sum(h * wc_r[...], axis=1, keepdims=True) + bc_r[0]
        o[...] = 1.0 / (1.0 + jnp.exp(-logit))

    node = lambda w: pl.BlockSpec((BN, w), lambda i: (i, 0))
    full = lambda a, b: pl.BlockSpec((a, b), lambda i: (0, 0))
    return pl.pallas_call(
        body,
        grid=(GRID,),
        in_specs=[node(H), node(H), node(CW), node(CW),
                  full(H, H), full(1, H), full(H, H), full(1, H),
                  pl.BlockSpec(memory_space=pltpu.SMEM)],
        out_specs=node(1),
        out_shape=jax.ShapeDtypeStruct((NP, 1), jnp.float32),
    )(gf, hf, c0, c1, W2l, b2l, W2r, Wc, bc)


def kernel(x, edge_index, W1l, b1l, W1r, W2l, b2l, W2r, Wc, bc):
    src = edge_index[0]
    dst = edge_index[1]
    z32 = jnp.zeros((DR, W), jnp.bfloat16)
    z8 = jnp.zeros((DR, CW), jnp.float32)
    o8 = jnp.ones((CC_, CW), jnp.float32)

    cnt = _counts(dst, z8, o8)
    af = _agg(x.astype(jnp.bfloat16).reshape(N * 2, W), 2, 2, src, dst, z32)
    hf = _dense1(af, cnt[0], cnt[1], x, W1l, b1l.reshape(1, H), W1r)
    gf = _agg(hf.reshape(NP * 4, W), 4, 4, src, dst, z32)
    out = _dense2(gf, hf, cnt[0], cnt[1],
                  W2l, b2l.reshape(1, H), W2r, Wc, bc.reshape(1))
    return out[:N]


# speedup vs baseline: 1.2531x; 1.2531x over previous
import functools

import jax
import jax.numpy as jnp
from jax import lax
from jax.experimental import pallas as pl
from jax.experimental.pallas import tpu as pltpu
from jax.experimental.pallas import tpu_sc as plsc

N = 50000
NP = 50176
E = 800000
D = 64
H = 128
W = 32

NC = 2
NS = 16
C = 80
EPT = E // NS
NITER = EPT // C
RPT = NP // NS
DR = 112
NDR = RPT // DR

CW = 8
CC_ = 40
CK = 5
CEPW = E // (NC * NS)
CNG = CEPW // (CC_ * CK)

K = 8
NG = (NITER - 1) // K

BN = 784
GRID = NP // BN

_mesh = lambda: plsc.VectorSubcoreMesh(core_axis_name="c", subcore_axis_name="s")


def _counts(dst, z8, o8):
    @functools.partial(
        pl.kernel,
        mesh=_mesh(),
        compiler_params=pltpu.CompilerParams(use_tc_tiling_on_sc=False),
        out_type=jax.ShapeDtypeStruct((NC, NP, CW), jnp.float32),
        scratch_types=[
            pltpu.VMEM((CK, CC_), jnp.int32),
            pltpu.VMEM((CC_, CW), jnp.float32),
            pltpu.VMEM((DR, CW), jnp.float32),
            pltpu.VMEM_SHARED((NP, CW), jnp.float32),
            pltpu.SemaphoreType.DMA,
        ],
    )
    def k(dstc_h, z8_h, o8_h, cnt_h, cidx, ones, cbuf, cacc, ssem):
        c = lax.axis_index("c")
        s = lax.axis_index("s")
        pltpu.sync_copy(o8_h, ones)
        pltpu.sync_copy(z8_h, cbuf)
        for j in range(NDR):
            pltpu.sync_copy(cbuf, cacc.at[pl.ds(s * RPT + j * DR, DR)])
        plsc.subcore_barrier()
        for cc in range(NC):
            @pl.when(c == cc)
            def _():
                def body(g, carry):
                    pltpu.sync_copy(dstc_h.at[s, cc, g], cidx)
                    sds = [pltpu.async_copy(ones, cacc.at[cidx.at[j]],
                                            ssem, add=True)
                           for j in range(CK)]
                    for d in sds:
                        d.wait()
                    return carry

                lax.fori_loop(0, CNG, body, 0)
                plsc.subcore_barrier()
                for j in range(NDR):
                    r0 = s * RPT + j * DR
                    pltpu.sync_copy(cacc.at[pl.ds(r0, DR)], cbuf)
                    pltpu.sync_copy(cbuf, cnt_h.at[cc, pl.ds(r0, DR)])

    return k(dst.reshape(NS, NC, CNG, CK, CC_), z8, o8)


WB = 64
NG1 = 312 // K


def _agg(table, F, spec, src, dst, z64):
    @functools.partial(
        pl.kernel,
        mesh=_mesh(),
        compiler_params=pltpu.CompilerParams(use_tc_tiling_on_sc=False),
        out_type=jax.ShapeDtypeStruct((NP, H), jnp.bfloat16),
        scratch_types=[
            pltpu.VMEM((2, K, C), jnp.int32),
            pltpu.VMEM((2, K, C), jnp.int32),
            pltpu.VMEM((K, C), jnp.int32),
            pltpu.VMEM((K, C, WB), jnp.bfloat16),
            pltpu.VMEM((DR, WB), jnp.bfloat16),
            pltpu.VMEM_SHARED((NP, WB), jnp.bfloat16),
            pltpu.SemaphoreType.DMA((K,)),
            pltpu.SemaphoreType.DMA((K,)),
        ],
    )
    def k(tbl_h, src_h, dst_h, z64_h, out_h,
          sidx2, didx2, sadj, rows2, dbuf, acc, gsem, ssem):
        c = lax.axis_index("c")
        s = lax.axis_index("s")

        def wait_slot(j, prev_p):
            pltpu.make_async_copy(rows2.at[j], acc.at[didx2.at[prev_p, j]],
                                  ssem.at[j]).wait()

        for cc in range(NC):
            ci, lo, ng, tail = spec[cc]

            @pl.when(c == cc)
            def _():
                pltpu.sync_copy(z64_h, dbuf)
                for j in range(NDR):
                    pltpu.sync_copy(dbuf, acc.at[pl.ds(s * RPT + j * DR, DR)])
                plsc.subcore_barrier()

                def gidx(p):
                    if F == 1:
                        return sidx2.at[p]
                    for j in range(K):
                        for u in range(C // 16):
                            v = sidx2[p, j, pl.ds(u * 16, 16)]
                            sadj[j, pl.ds(u * 16, 16)] = v * F + ci
                    return sadj

                def gbody(g, p, first):
                    pltpu.sync_copy(src_h.at[s, pl.ds(lo + K * g, K)],
                                    sidx2.at[p])
                    pltpu.sync_copy(dst_h.at[s, pl.ds(lo + K * g, K)],
                                    didx2.at[p])
                    gi = gidx(p)
                    gds = []
                    for j in range(K):
                        if not first:
                            wait_slot(j, 1 - p)
                        gds.append(pltpu.async_copy(
                            tbl_h.at[gi.at[j]], rows2.at[j], gsem.at[j]))
                    for j in range(K):
                        gds[j].wait()
                        pltpu.async_copy(rows2.at[j],
                                         acc.at[didx2.at[p, j]],
                                         ssem.at[j], add=True)

                gbody(0, 0, True)

                def pair(t, carry):
                    gbody(2 * t + 1, 1, False)
                    gbody(2 * t + 2, 0, False)
                    return carry

                lax.fori_loop(0, (ng - 1) // 2, pair, 0)
                if (ng - 1) % 2 == 1:
                    gbody(ng - 1, 1, False)
                last_p = (ng - 1) % 2
                for j in range(K):
                    wait_slot(j, last_p)
                if tail:
                    tc_ = lo + K * ng
                    pltpu.sync_copy(src_h.at[s, pl.ds(tc_, 1)],
                                    sidx2.at[0, pl.ds(0, 1)])
                    pltpu.sync_copy(dst_h.at[s, pl.ds(tc_, 1)],
                                    didx2.at[0, pl.ds(0, 1)])
                    if F == 1:
                        ti = sidx2.at[0, 0]
                    else:
                        for u in range(C // 16):
                            v = sidx2[0, 0, pl.ds(u * 16, 16)]
                            sadj[0, pl.ds(u * 16, 16)] = v * F + ci
                        ti = sadj.at[0]
                    pltpu.async_copy(tbl_h.at[ti], rows2.at[0],
                                     gsem.at[0]).wait()
                    pltpu.sync_copy(rows2.at[0], acc.at[didx2.at[0, 0]],
                                    add=True)
                plsc.subcore_barrier()
                for j in range(NDR):
                    r0 = s * RPT + j * DR
                    pltpu.sync_copy(acc.at[pl.ds(r0, DR)], dbuf)
                    pltpu.sync_copy(
                        dbuf, out_h.at[pl.ds(r0, DR), pl.ds(ci * WB, WB)])

    return k(table, src.reshape(NS, NITER, C), dst.reshape(NS, NITER, C), z64)


def _dense1(af, c0, c1, x, W1l, b1l, W1r):
    def body(af_r, c0_r, c1_r, x_r, wl_r, bl_r, wr_r, o):
        cnt = c0_r[:, 0:1] + c1_r[:, 0:1]
        recip = 1.0 / jnp.maximum(cnt, 1.0)
        a = af_r[:, :D].astype(jnp.float32) + af_r[:, D:].astype(jnp.float32)
        m = lax.dot_general(a, wl_r[...],
                            (((1,), (1,)), ((), ())),
                            preferred_element_type=jnp.float32)
        sf = lax.dot_general(x_r[...], wr_r[...],
                             (((1,), (1,)), ((), ())),
                             preferred_element_type=jnp.float32)
        o[...] = jnp.maximum(m * recip + bl_r[...] + sf, 0.0).astype(jnp.bfloat16)

    node = lambda w: pl.BlockSpec((BN, w), lambda i: (i, 0))
    full = lambda a, b: pl.BlockSpec((a, b), lambda i: (0, 0))
    return pl.pallas_call(
        body,
        grid=(GRID,),
        in_specs=[node(H), node(CW), node(CW), node(D),
                  full(H, D), full(1, H), full(H, D)],
        out_specs=node(H),
        out_shape=jax.ShapeDtypeStruct((NP, H), jnp.bfloat16),
    )(af, c0, c1, x, W1l, b1l, W1r)


def _dense2(gf, hf, c0, c1, W2l, b2l, W2r, Wc, bc):
    def body(gf_r, hf_r, c0_r, c1_r, wl_r, bl_r, wr_r, wc_r, bc_r, o):
        cnt = c0_r[:, 0:1] + c1_r[:, 0:1]
        recip = 1.0 / jnp.maximum(cnt, 1.0)
        m = lax.dot_general(gf_r[...].astype(jnp.float32), wl_r[...],
                            (((1,), (1,)), ((), ())),
                            preferred_element_type=jnp.float32)
        sf = lax.dot_general(hf_r[...].astype(jnp.float32), wr_r[...],
                             (((1,), (1,)), ((), ())),
                             preferred_element_type=jnp.float32)
        h = jnp.maximum(m * recip + bl_r[...] + sf, 0.0)
        logit = jnp.sum(h * wc_r[...], axis=1, keepdims=True) + bc_r[0]
        o[...] = 1.0 / (1.0 + jnp.exp(-logit))

    node = lambda w: pl.BlockSpec((BN, w), lambda i: (i, 0))
    full = lambda a, b: pl.BlockSpec((a, b), lambda i: (0, 0))
    return pl.pallas_call(
        body,
        grid=(GRID,),
        in_specs=[node(H), node(H), node(CW), node(CW),
                  full(H, H), full(1, H), full(H, H), full(1, H),
                  pl.BlockSpec(memory_space=pltpu.SMEM)],
        out_specs=node(1),
        out_shape=jax.ShapeDtypeStruct((NP, 1), jnp.float32),
    )(gf, hf, c0, c1, W2l, b2l, W2r, Wc, bc)


def kernel(x, edge_index, W1l, b1l, W1r, W2l, b2l, W2r, Wc, bc):
    src = edge_index[0]
    dst = edge_index[1]
    z64 = jnp.zeros((DR, WB), jnp.bfloat16)
    z8 = jnp.zeros((DR, CW), jnp.float32)
    o8 = jnp.ones((CC_, CW), jnp.float32)

    cnt = _counts(dst, z8, o8)
    af = _agg(x.astype(jnp.bfloat16), 1,
              [(0, 0, NG1, False), (1, 312, NG1, True)], src, dst, z64)
    hf = _dense1(af, cnt[0], cnt[1], x, W1l, b1l.reshape(1, H), W1r)
    gf = _agg(hf.reshape(NP * 2, WB), 2,
              [(0, 0, NG, True), (1, 0, NG, True)], src, dst, z64)
    out = _dense2(gf, hf, cnt[0], cnt[1],
                  W2l, b2l.reshape(1, H), W2r, Wc, bc.reshape(1))
    return out[:N]


# speedup vs baseline: 1.2705x; 1.0140x over previous
import functools

import jax
import jax.numpy as jnp
from jax import lax
from jax.experimental import pallas as pl
from jax.experimental.pallas import tpu as pltpu
from jax.experimental.pallas import tpu_sc as plsc

N = 50000
NP = 50176
E = 800000
D = 64
H = 128
W = 32

NC = 2
NS = 16
C = 80
EPT = E // NS
NITER = EPT // C
RPT = NP // NS
DR = 112
NDR = RPT // DR

CW = 8
CC_ = 40
CK = 5
CEPW = E // (NC * NS)
CNG = CEPW // (CC_ * CK)

K = 8
NG = (NITER - 1) // K

BN = 784
GRID = NP // BN

_mesh = lambda: plsc.VectorSubcoreMesh(core_axis_name="c", subcore_axis_name="s")


def _counts(dst, z8, o8):
    @functools.partial(
        pl.kernel,
        mesh=_mesh(),
        compiler_params=pltpu.CompilerParams(use_tc_tiling_on_sc=False),
        out_type=jax.ShapeDtypeStruct((NC, NP, CW), jnp.float32),
        scratch_types=[
            pltpu.VMEM((2, CK, CC_), jnp.int32),
            pltpu.VMEM((CC_, CW), jnp.float32),
            pltpu.VMEM((DR, CW), jnp.float32),
            pltpu.VMEM_SHARED((NP, CW), jnp.float32),
            pltpu.SemaphoreType.DMA((CK,)),
        ],
    )
    def k(dstc_h, z8_h, o8_h, cnt_h, cidx, ones, cbuf, cacc, ssem):
        c = lax.axis_index("c")
        s = lax.axis_index("s")
        pltpu.sync_copy(o8_h, ones)
        pltpu.sync_copy(z8_h, cbuf)
        for j in range(NDR):
            pltpu.sync_copy(cbuf, cacc.at[pl.ds(s * RPT + j * DR, DR)])
        plsc.subcore_barrier()

        def wait_slot(j, prev_p):
            pltpu.make_async_copy(ones, cacc.at[cidx.at[prev_p, j]],
                                  ssem.at[j]).wait()

        for cc in range(NC):
            @pl.when(c == cc)
            def _():
                def gbody(g, p, first):
                    pltpu.sync_copy(dstc_h.at[s, cc, g], cidx.at[p])
                    for j in range(CK):
                        if not first:
                            wait_slot(j, 1 - p)
                        pltpu.async_copy(ones, cacc.at[cidx.at[p, j]],
                                         ssem.at[j], add=True)

                gbody(0, 0, True)

                def pair(t, carry):
                    gbody(2 * t + 1, 1, False)
                    gbody(2 * t + 2, 0, False)
                    return carry

                lax.fori_loop(0, (CNG - 1) // 2, pair, 0)
                if (CNG - 1) % 2 == 1:
                    gbody(CNG - 1, 1, False)
                for j in range(CK):
                    wait_slot(j, (CNG - 1) % 2)
                plsc.subcore_barrier()
                for j in range(NDR):
                    r0 = s * RPT + j * DR
                    pltpu.sync_copy(cacc.at[pl.ds(r0, DR)], cbuf)
                    pltpu.sync_copy(cbuf, cnt_h.at[cc, pl.ds(r0, DR)])

    return k(dst.reshape(NS, NC, CNG, CK, CC_), z8, o8)


WB = 64
NG1 = 312 // K


def _agg(table, F, spec, src, dst, z64):
    @functools.partial(
        pl.kernel,
        mesh=_mesh(),
        compiler_params=pltpu.CompilerParams(use_tc_tiling_on_sc=False),
        out_type=jax.ShapeDtypeStruct((NP, H), jnp.bfloat16),
        scratch_types=[
            pltpu.VMEM((2, K, C), jnp.int32),
            pltpu.VMEM((2, K, C), jnp.int32),
            pltpu.VMEM((K, C), jnp.int32),
            pltpu.VMEM((K, C, WB), jnp.bfloat16),
            pltpu.VMEM((DR, WB), jnp.bfloat16),
            pltpu.VMEM_SHARED((NP, WB), jnp.bfloat16),
            pltpu.SemaphoreType.DMA((K,)),
            pltpu.SemaphoreType.DMA((K,)),
        ],
    )
    def k(tbl_h, src_h, dst_h, z64_h, out_h,
          sidx2, didx2, sadj, rows2, dbuf, acc, gsem, ssem):
        c = lax.axis_index("c")
        s = lax.axis_index("s")

        def wait_slot(j, prev_p):
            pltpu.make_async_copy(rows2.at[j], acc.at[didx2.at[prev_p, j]],
                                  ssem.at[j]).wait()

        for cc in range(NC):
            ci, lo, ng, tail = spec[cc]

            @pl.when(c == cc)
            def _():
                pltpu.sync_copy(z64_h, dbuf)
                for j in range(NDR):
                    pltpu.sync_copy(dbuf, acc.at[pl.ds(s * RPT + j * DR, DR)])
                plsc.subcore_barrier()

                def gidx(p):
                    if F == 1:
                        return sidx2.at[p]
                    for j in range(K):
                        for u in range(C // 16):
                            v = sidx2[p, j, pl.ds(u * 16, 16)]
                            sadj[j, pl.ds(u * 16, 16)] = v * F + ci
                    return sadj

                def gbody(g, p, first):
                    pltpu.sync_copy(src_h.at[s, pl.ds(lo + K * g, K)],
                                    sidx2.at[p])
                    pltpu.sync_copy(dst_h.at[s, pl.ds(lo + K * g, K)],
                                    didx2.at[p])
                    gi = gidx(p)
                    gds = []
                    for j in range(K):
                        if not first:
                            wait_slot(j, 1 - p)
                        gds.append(pltpu.async_copy(
                            tbl_h.at[gi.at[j]], rows2.at[j], gsem.at[j]))
                    for j in range(K):
                        gds[j].wait()
                        pltpu.async_copy(rows2.at[j],
                                         acc.at[didx2.at[p, j]],
                                         ssem.at[j], add=True)

                gbody(0, 0, True)

                def pair(t, carry):
                    gbody(2 * t + 1, 1, False)
                    gbody(2 * t + 2, 0, False)
                    return carry

                lax.fori_loop(0, (ng - 1) // 2, pair, 0)
                if (ng - 1) % 2 == 1:
                    gbody(ng - 1, 1, False)
                last_p = (ng - 1) % 2
                for j in range(K):
                    wait_slot(j, last_p)
                if tail:
                    tc_ = lo + K * ng
                    pltpu.sync_copy(src_h.at[s, pl.ds(tc_, 1)],
                                    sidx2.at[0, pl.ds(0, 1)])
                    pltpu.sync_copy(dst_h.at[s, pl.ds(tc_, 1)],
                                    didx2.at[0, pl.ds(0, 1)])
                    if F == 1:
                        ti = sidx2.at[0, 0]
                    else:
                        for u in range(C // 16):
                            v = sidx2[0, 0, pl.ds(u * 16, 16)]
                            sadj[0, pl.ds(u * 16, 16)] = v * F + ci
                        ti = sadj.at[0]
                    pltpu.async_copy(tbl_h.at[ti], rows2.at[0],
                                     gsem.at[0]).wait()
                    pltpu.sync_copy(rows2.at[0], acc.at[didx2.at[0, 0]],
                                    add=True)
                plsc.subcore_barrier()
                for j in range(NDR):
                    r0 = s * RPT + j * DR
                    pltpu.sync_copy(acc.at[pl.ds(r0, DR)], dbuf)
                    pltpu.sync_copy(
                        dbuf, out_h.at[pl.ds(r0, DR), pl.ds(ci * WB, WB)])

    return k(table, src.reshape(NS, NITER, C), dst.reshape(NS, NITER, C), z64)


def _dense1(af, c0, c1, x, W1l, b1l, W1r):
    def body(af_r, c0_r, c1_r, x_r, wl_r, bl_r, wr_r, o):
        cnt = c0_r[:, 0:1] + c1_r[:, 0:1]
        recip = 1.0 / jnp.maximum(cnt, 1.0)
        a = af_r[:, :D].astype(jnp.float32) + af_r[:, D:].astype(jnp.float32)
        m = lax.dot_general(a, wl_r[...],
                            (((1,), (1,)), ((), ())),
                            preferred_element_type=jnp.float32)
        sf = lax.dot_general(x_r[...], wr_r[...],
                             (((1,), (1,)), ((), ())),
                             preferred_element_type=jnp.float32)
        o[...] = jnp.maximum(m * recip + bl_r[...] + sf, 0.0).astype(jnp.bfloat16)

    node = lambda w: pl.BlockSpec((BN, w), lambda i: (i, 0))
    full = lambda a, b: pl.BlockSpec((a, b), lambda i: (0, 0))
    return pl.pallas_call(
        body,
        grid=(GRID,),
        in_specs=[node(H), node(CW), node(CW), node(D),
                  full(H, D), full(1, H), full(H, D)],
        out_specs=node(H),
        out_shape=jax.ShapeDtypeStruct((NP, H), jnp.bfloat16),
    )(af, c0, c1, x, W1l, b1l, W1r)


def _dense2(gf, hf, c0, c1, W2l, b2l, W2r, Wc, bc):
    def body(gf_r, hf_r, c0_r, c1_r, wl_r, bl_r, wr_r, wc_r, bc_r, o):
        cnt = c0_r[:, 0:1] + c1_r[:, 0:1]
        recip = 1.0 / jnp.maximum(cnt, 1.0)
        m = lax.dot_general(gf_r[...].astype(jnp.float32), wl_r[...],
                            (((1,), (1,)), ((), ())),
                            preferred_element_type=jnp.float32)
        sf = lax.dot_general(hf_r[...].astype(jnp.float32), wr_r[...],
                             (((1,), (1,)), ((), ())),
                             preferred_element_type=jnp.float32)
        h = jnp.maximum(m * recip + bl_r[...] + sf, 0.0)
        logit = jnp.sum(h * wc_r[...], axis=1, keepdims=True) + bc_r[0]
        o[...] = 1.0 / (1.0 + jnp.exp(-logit))

    node = lambda w: pl.BlockSpec((BN, w), lambda i: (i, 0))
    full = lambda a, b: pl.BlockSpec((a, b), lambda i: (0, 0))
    return pl.pallas_call(
        body,
        grid=(GRID,),
        in_specs=[node(H), node(H), node(CW), node(CW),
                  full(H, H), full(1, H), full(H, H), full(1, H),
                  pl.BlockSpec(memory_space=pltpu.SMEM)],
        out_specs=node(1),
        out_shape=jax.ShapeDtypeStruct((NP, 1), jnp.float32),
    )(gf, hf, c0, c1, W2l, b2l, W2r, Wc, bc)


def kernel(x, edge_index, W1l, b1l, W1r, W2l, b2l, W2r, Wc, bc):
    src = edge_index[0]
    dst = edge_index[1]
    z64 = jnp.zeros((DR, WB), jnp.bfloat16)
    z8 = jnp.zeros((DR, CW), jnp.float32)
    o8 = jnp.ones((CC_, CW), jnp.float32)

    cnt = _counts(dst, z8, o8)
    af = _agg(x.astype(jnp.bfloat16), 1,
              [(0, 0, NG1, False), (1, 312, NG1, True)], src, dst, z64)
    hf = _dense1(af, cnt[0], cnt[1], x, W1l, b1l.reshape(1, H), W1r)
    gf = _agg(hf.reshape(NP * 2, WB), 2,
              [(0, 0, NG, True), (1, 0, NG, True)], src, dst, z64)
    out = _dense2(gf, hf, cnt[0], cnt[1],
                  W2l, b2l.reshape(1, H), W2r, Wc, bc.reshape(1))
    return out[:N]
